# baseline (speedup 1.0000x reference)
import functools

import jax
import jax.numpy as jnp
from jax import lax
from jax.experimental import pallas as pl
from jax.experimental.pallas import tpu as pltpu
from jax.experimental.pallas import tpu_sc as plsc

VOCAB = 1000000
D = 64
L = 50
LC = 51
NG = 104
NLANE = 16
NC = 2
NS = 16
NW = NC * NS
BATCH = 16384
PAIRS = BATCH // 2
PPT = PAIRS // NW


def _shuf(x, perm):
    return x.at[perm].get(mode="promise_in_bounds")


def _splat_sum(x, lane):
    for r in (8, 4, 2, 1):
        x = x + _shuf(x, lane ^ r)
    return x


def _combine(R, rbase, hoff, out_v, prow, ocol):
    def row(k, q):
        start = pl.multiple_of(hoff(rbase + k) + NLANE * q, NLANE)
        return R[rbase + k, pl.ds(start, NLANE)]

    c = [row(L, q) for q in range(4)]
    lane = lax.iota(jnp.int32, NLANE)
    zero = jnp.zeros((NLANE,), jnp.float32)
    acc = [zero] * 4
    wacc = zero
    for k in range(L):
        x = [row(k, q) for q in range(4)]
        s = None
        for q in range(4):
            d = x[q] - c[q]
            s = d * d if s is None else s + d * d
        w = jnp.exp(_splat_sum(s, lane) * -0.5)
        wacc = wacc + w
        for q in range(4):
            acc[q] = acc[q] + w * x[q]
    inv = 1.0 / (wacc + 1e-8)
    for q in range(4):
        out_v[prow, pl.ds(ocol + NLANE * q, NLANE)] = acc[q] * inv


@functools.partial(
    pl.kernel,
    out_type=jax.ShapeDtypeStruct((PAIRS, 2 * D), jnp.float32),
    mesh=plsc.VectorSubcoreMesh(core_axis_name="c", subcore_axis_name="s"),
    scratch_types=[
        pltpu.VMEM((PPT, 128), jnp.int32),
        pltpu.VMEM((PPT, 2 * D), jnp.float32),
        pltpu.VMEM((NG, 2 * D), jnp.float32),
        pltpu.VMEM((NG, 2 * D), jnp.float32),
        pltpu.VMEM((NG, 2 * D), jnp.float32),
        pltpu.VMEM((NG, 2 * D), jnp.float32),
        pltpu.SemaphoreType.DMA,
        pltpu.SemaphoreType.DMA,
        pltpu.SemaphoreType.DMA,
        pltpu.SemaphoreType.DMA,
    ],
)
def _krembed(idx_hbm, table_hbm, out_hbm, idx_v, out_v,
             buf0, buf1, buf2, buf3, sem0, sem1, sem2, sem3):
    wid = lax.axis_index("s") * NC + lax.axis_index("c")
    base = wid * PPT
    bufs = [buf0, buf1, buf2, buf3]
    sems = [sem0, sem1, sem2, sem3]
    NBUF = 4
    pltpu.sync_copy(idx_hbm.at[pl.ds(base, PPT)], idx_v)

    def fire(p, u):
        pltpu.async_copy(table_hbm.at[idx_v.at[p, pl.ds(0, NG)]],
                         bufs[u], sems[u])

    def drain(u):
        pltpu.make_async_copy(table_hbm.at[pl.ds(0, NG)],
                              bufs[u], sems[u]).wait()

    def combine2(buf, p):
        hv = idx_v[p, pl.ds(96, NLANE)]
        hw = [hv[8], hv[9], hv[10], hv[11]]

        def hoff(j02):
            return ((hw[j02 // 32] >> (j02 % 32)) & 1) << 6

        _combine(buf, 0, hoff, out_v, p, 0)
        _combine(buf, LC, hoff, out_v, p, D)

    for u in range(NBUF - 1):
        fire(u, u)

    def body(i, carry):
        p = i * NBUF
        for u in range(NBUF):
            pu = p + u

            @pl.when(pu + NBUF - 1 < PPT)
            def _(pu=pu, u=u):
                fire(pu + NBUF - 1, (u + NBUF - 1) % NBUF)

            drain(u)
            combine2(bufs[u], pu)
        return carry

    lax.fori_loop(0, PPT // NBUF, body, 0)
    pltpu.sync_copy(out_v, out_hbm.at[pl.ds(base, PPT)])


def kernel(context, center, embedding_weights):
    i102 = jnp.concatenate([context, center[:, None]],
                           axis=1).reshape(PAIRS, 2 * LC)
    slots = jnp.pad(i102 >> 1, ((0, 0), (0, NG - 2 * LC)))
    hbits = jnp.pad(i102 & 1, ((0, 0), (0, 128 - 2 * LC)))
    shift = jnp.arange(32, dtype=jnp.int32)[None, None, :]
    hw = jnp.sum(hbits.reshape(PAIRS, 4, 32) << shift,
                 axis=2, dtype=jnp.int32)
    idx = jnp.concatenate(
        [slots, hw, jnp.zeros((PAIRS, 128 - NG - 4), jnp.int32)], axis=1)
    table2 = embedding_weights.reshape(VOCAB // 2, 2 * D)
    packed = _krembed(idx, table2)
    return packed.reshape(BATCH, D)


# speedup vs baseline: 1.1638x; 1.1638x over previous
import functools

import jax
import jax.numpy as jnp
from jax import lax
from jax.experimental import pallas as pl
from jax.experimental.pallas import tpu as pltpu
from jax.experimental.pallas import tpu_sc as plsc

VOCAB = 1000000
D = 64
L = 50
LC = 51
NG = 104
NLANE = 16
NC = 2
NS = 16
NW = NC * NS
BATCH = 16384
PAIRS = BATCH // 2
PPT = PAIRS // NW
RW = 1024
RG = (VOCAB + RW - 1) // RW
NSLOT = RG * (RW // 2)


@functools.partial(
    pl.pallas_call,
    grid=(RG,),
    in_specs=[pl.BlockSpec((D, RW), lambda k: (0, k))],
    out_specs=pl.BlockSpec((RW // 2, 2 * D), lambda k: (k, 0)),
    out_shape=jax.ShapeDtypeStruct((NSLOT, 2 * D), jnp.float32),
)
def _repack(tT_ref, o_ref):
    xt = tT_ref[...].T
    o_ref[...] = jnp.concatenate([xt[: RW // 2], xt[RW // 2:]], axis=1)


def _shuf(x, perm):
    return x.at[perm].get(mode="promise_in_bounds")


def _splat_sum(x, lane):
    for r in (8, 4, 2, 1):
        x = x + _shuf(x, lane ^ r)
    return x


def _combine(R, rbase, out_v, prow, ocol):
    c = [R[rbase + L, pl.ds(NLANE * q, NLANE)] for q in range(4)]
    lane = lax.iota(jnp.int32, NLANE)
    zero = jnp.zeros((NLANE,), jnp.float32)
    acc = [zero] * 4
    wacc = zero
    for k in range(L):
        x = [R[rbase + k, pl.ds(NLANE * q, NLANE)] for q in range(4)]
        s = None
        for q in range(4):
            d = x[q] - c[q]
            s = d * d if s is None else s + d * d
        w = jnp.exp(_splat_sum(s, lane) * -0.5)
        wacc = wacc + w
        for q in range(4):
            acc[q] = acc[q] + w * x[q]
    inv = 1.0 / (wacc + 1e-8)
    for q in range(4):
        out_v[prow, pl.ds(ocol + NLANE * q, NLANE)] = acc[q] * inv


@functools.partial(
    pl.kernel,
    out_type=jax.ShapeDtypeStruct((PAIRS, 2 * D), jnp.float32),
    mesh=plsc.VectorSubcoreMesh(core_axis_name="c", subcore_axis_name="s"),
    compiler_params=pltpu.CompilerParams(use_tc_tiling_on_sc=False),
    scratch_types=[
        pltpu.VMEM((PPT, 128), jnp.int32),
        pltpu.VMEM((PPT, 2 * D), jnp.float32),
        pltpu.VMEM((NG, D), jnp.float32),
        pltpu.VMEM((NG, D), jnp.float32),
        pltpu.VMEM((NG, D), jnp.float32),
        pltpu.VMEM((NG, D), jnp.float32),
        pltpu.SemaphoreType.DMA,
        pltpu.SemaphoreType.DMA,
        pltpu.SemaphoreType.DMA,
        pltpu.SemaphoreType.DMA,
    ],
)
def _krembed(idx_hbm, table_hbm, out_hbm, idx_v, out_v,
             buf0, buf1, buf2, buf3, sem0, sem1, sem2, sem3):
    wid = lax.axis_index("s") * NC + lax.axis_index("c")
    base = wid * PPT
    bufs = [buf0, buf1, buf2, buf3]
    sems = [sem0, sem1, sem2, sem3]
    NBUF = 4
    pltpu.sync_copy(idx_hbm.at[pl.ds(base, PPT)], idx_v)

    def fire(p, u):
        pltpu.async_copy(table_hbm.at[idx_v.at[p, pl.ds(0, NG)]],
                         bufs[u], sems[u])

    def drain(u):
        pltpu.make_async_copy(table_hbm.at[pl.ds(0, NG)],
                              bufs[u], sems[u]).wait()

    def combine2(buf, p):
        _combine(buf, 0, out_v, p, 0)
        _combine(buf, LC + 1, out_v, p, D)

    for u in range(NBUF - 1):
        fire(u, u)

    def body(i, carry):
        p = i * NBUF
        for u in range(NBUF):
            pu = p + u

            @pl.when(pu + NBUF - 1 < PPT)
            def _(pu=pu, u=u):
                fire(pu + NBUF - 1, (u + NBUF - 1) % NBUF)

            drain(u)
            combine2(bufs[u], pu)
        return carry

    lax.fori_loop(0, PPT // NBUF, body, 0)
    pltpu.sync_copy(out_v, out_hbm.at[pl.ds(base, PPT)])


def kernel(context, center, embedding_weights):
    i102 = jnp.concatenate([context, center[:, None]],
                           axis=1).reshape(PAIRS, 2 * LC)
    rem = i102 % RW
    r102 = ((i102 // RW) * RW + ((rem & (RW // 2 - 1)) << 1)
            + (rem >> 9)).astype(jnp.int32)
    rows = jnp.concatenate(
        [r102[:, :LC], r102[:, L:LC],
         r102[:, LC:], r102[:, 2 * LC - 1:]],
        axis=1)
    idx = jnp.pad(rows, ((0, 0), (0, 128 - NG)))
    table3 = _repack(embedding_weights.T).reshape(2 * NSLOT, D)
    packed = _krembed(idx, table3)
    return packed.reshape(BATCH, D)


# speedup vs baseline: 1.6356x; 1.4054x over previous
import functools

import jax
import jax.numpy as jnp
from jax import lax
from jax.experimental import pallas as pl
from jax.experimental.pallas import tpu as pltpu
from jax.experimental.pallas import tpu_sc as plsc

VOCAB = 1000000
D = 64
L = 50
LC = 51
NH = 56
NLANE = 16
NC = 2
NS = 16
NW = NC * NS
BATCH = 16384
PAIRS = BATCH // 2
PPT = PAIRS // NW
RW = 4096
RG = (VOCAB + RW - 1) // RW
NSLOT = RG * (RW // 2)


@functools.partial(
    pl.pallas_call,
    grid=(RG,),
    in_specs=[pl.BlockSpec((D, RW), lambda k: (0, k))],
    out_specs=pl.BlockSpec((RW // 2, 2 * D), lambda k: (k, 0)),
    out_shape=jax.ShapeDtypeStruct((NSLOT, 2 * D), jnp.float32),
)
def _repack(tT_ref, o_ref):
    xt = tT_ref[...].T
    o_ref[...] = jnp.concatenate([xt[: RW // 2], xt[RW // 2:]], axis=1)


def _shuf(x, perm):
    return x.at[perm].get(mode="promise_in_bounds")


def _splat_sum(x, lane):
    for r in (8, 4, 2, 1):
        x = x + _shuf(x, lane ^ r)
    return x


def _combine(R, rbase, out_v, prow, ocol):
    c = [R[rbase + L, pl.ds(NLANE * q, NLANE)] for q in range(4)]
    lane = lax.iota(jnp.int32, NLANE)
    zero = jnp.zeros((NLANE,), jnp.float32)
    acc = [zero] * 4
    wacc = zero
    for k in range(L):
        x = [R[rbase + k, pl.ds(NLANE * q, NLANE)] for q in range(4)]
        s = None
        for q in range(4):
            d = x[q] - c[q]
            s = d * d if s is None else s + d * d
        w = jnp.exp(_splat_sum(s, lane) * -0.5)
        wacc = wacc + w
        for q in range(4):
            acc[q] = acc[q] + w * x[q]
    inv = 1.0 / (wacc + 1e-8)
    for q in range(4):
        out_v[prow, pl.ds(ocol + NLANE * q, NLANE)] = acc[q] * inv


@functools.partial(
    pl.kernel,
    out_type=jax.ShapeDtypeStruct((PAIRS, 2 * D), jnp.float32),
    mesh=plsc.VectorSubcoreMesh(core_axis_name="c", subcore_axis_name="s"),
    compiler_params=pltpu.CompilerParams(use_tc_tiling_on_sc=False),
    scratch_types=(
        [pltpu.VMEM((PPT, 128), jnp.int32),
         pltpu.VMEM((PPT, 2 * D), jnp.float32)]
        + [pltpu.VMEM((NH, D), jnp.float32)] * 8
        + [pltpu.SemaphoreType.DMA] * 8
    ),
)
def _krembed(idx_hbm, table_hbm, out_hbm, idx_v, out_v, *bufsems):
    wid = lax.axis_index("s") * NC + lax.axis_index("c")
    base = wid * PPT
    bufs = bufsems[:8]
    sems = bufsems[8:]
    NBUF = 8
    NU = 2 * PPT
    pltpu.sync_copy(idx_hbm.at[pl.ds(base, PPT)], idx_v)

    def fire(u, t):
        p, off = u >> 1, (t & 1) * NH
        pltpu.async_copy(table_hbm.at[idx_v.at[p, pl.ds(off, NH)]],
                         bufs[t], sems[t])

    def drain(t):
        pltpu.make_async_copy(table_hbm.at[pl.ds(0, NH)],
                              bufs[t], sems[t]).wait()

    for t in range(NBUF - 1):
        fire(t, t)

    def body(i, carry):
        ub = i * NBUF
        for t in range(NBUF):
            u = ub + t

            @pl.when(u + NBUF - 1 < NU)
            def _(u=u, t=t):
                fire(u + NBUF - 1, (t + NBUF - 1) % NBUF)

            drain(t)
            _combine(bufs[t], 0, out_v, u >> 1, (t & 1) * D)
        return carry

    lax.fori_loop(0, NU // NBUF, body, 0)
    pltpu.sync_copy(out_v, out_hbm.at[pl.ds(base, PPT)])


def kernel(context, center, embedding_weights):
    i102 = jnp.concatenate([context, center[:, None]],
                           axis=1).reshape(PAIRS, 2 * LC)
    rem = i102 % RW
    r102 = ((i102 // RW) * RW + ((rem & (RW // 2 - 1)) << 1)
            + (rem >> 11)).astype(jnp.int32)
    pad0 = jnp.broadcast_to(r102[:, L:LC], (PAIRS, NH - LC))
    pad1 = jnp.broadcast_to(r102[:, 2 * LC - 1:], (PAIRS, NH - LC))
    rows = jnp.concatenate(
        [r102[:, :LC], pad0,
         r102[:, LC:], pad1],
        axis=1)
    idx = jnp.pad(rows, ((0, 0), (0, 128 - 2 * NH)))
    table3 = _repack(embedding_weights.T).reshape(2 * NSLOT, D)
    packed = _krembed(idx, table3)
    return packed.reshape(BATCH, D)


# speedup vs baseline: 1.7616x; 1.0771x over previous
import functools

import jax
import jax.numpy as jnp
from jax import lax
from jax.experimental import pallas as pl
from jax.experimental.pallas import tpu as pltpu
from jax.experimental.pallas import tpu_sc as plsc

VOCAB = 1000000
D = 64
L = 50
LC = 51
NH = 56
NLANE = 16
NC = 2
NS = 16
NW = NC * NS
BATCH = 16384
PAIRS = BATCH // 2
PPT = PAIRS // NW
RW = 8192
RG = (VOCAB + RW - 1) // RW
NSLOT = RG * (RW // 2)


@functools.partial(
    pl.pallas_call,
    grid=(RG,),
    in_specs=[pl.BlockSpec((D, RW), lambda k: (0, k))],
    out_specs=pl.BlockSpec((RW // 2, 2 * D), lambda k: (k, 0)),
    out_shape=jax.ShapeDtypeStruct((NSLOT, 2 * D), jnp.float32),
)
def _repack(tT_ref, o_ref):
    xt = tT_ref[...].T
    o_ref[...] = jnp.concatenate([xt[: RW // 2], xt[RW // 2:]], axis=1)


def _shuf(x, perm):
    return x.at[perm].get(mode="promise_in_bounds")


def _splat_sum(x, lane):
    for r in (8, 4, 2, 1):
        x = x + _shuf(x, lane ^ r)
    return x


def _combine(R, rbase, out_v, prow, ocol):
    c = [R[rbase + L, pl.ds(NLANE * q, NLANE)] for q in range(4)]
    lane = lax.iota(jnp.int32, NLANE)
    zero = jnp.zeros((NLANE,), jnp.float32)
    acc = [zero] * 4
    wacc = zero
    for k in range(L):
        x = [R[rbase + k, pl.ds(NLANE * q, NLANE)] for q in range(4)]
        s = None
        for q in range(4):
            d = x[q] - c[q]
            s = d * d if s is None else s + d * d
        w = jnp.exp(_splat_sum(s, lane) * -0.5)
        wacc = wacc + w
        for q in range(4):
            acc[q] = acc[q] + w * x[q]
    inv = 1.0 / (wacc + 1e-8)
    for q in range(4):
        out_v[prow, pl.ds(ocol + NLANE * q, NLANE)] = acc[q] * inv


@functools.partial(
    pl.kernel,
    out_type=jax.ShapeDtypeStruct((PAIRS, 2 * D), jnp.float32),
    mesh=plsc.VectorSubcoreMesh(core_axis_name="c", subcore_axis_name="s"),
    compiler_params=pltpu.CompilerParams(use_tc_tiling_on_sc=False),
    scratch_types=(
        [pltpu.VMEM((PPT, 128), jnp.int32),
         pltpu.VMEM((PPT, 2 * D), jnp.float32)]
        + [pltpu.VMEM((NH, D), jnp.float32)] * 8
        + [pltpu.SemaphoreType.DMA] * 8
    ),
)
def _krembed(idx_hbm, table_hbm, out_hbm, idx_v, out_v, *bufsems):
    wid = lax.axis_index("s") * NC + lax.axis_index("c")
    base = wid * PPT
    bufs = bufsems[:8]
    sems = bufsems[8:]
    NBUF = 8
    NU = 2 * PPT
    pltpu.sync_copy(idx_hbm.at[pl.ds(base, PPT)], idx_v)

    def fire(u, t):
        p, off = u >> 1, (t & 1) * NH
        pltpu.async_copy(table_hbm.at[idx_v.at[p, pl.ds(off, NH)]],
                         bufs[t], sems[t])

    def drain(t):
        pltpu.make_async_copy(table_hbm.at[pl.ds(0, NH)],
                              bufs[t], sems[t]).wait()

    for t in range(NBUF - 1):
        fire(t, t)

    def body(i, carry):
        ub = i * NBUF
        for t in range(NBUF):
            u = ub + t

            @pl.when(u + NBUF - 1 < NU)
            def _(u=u, t=t):
                fire(u + NBUF - 1, (t + NBUF - 1) % NBUF)

            drain(t)
            _combine(bufs[t], 0, out_v, u >> 1, (t & 1) * D)
        return carry

    lax.fori_loop(0, NU // NBUF, body, 0)
    pltpu.sync_copy(out_v, out_hbm.at[pl.ds(base, PPT)])


def kernel(context, center, embedding_weights):
    i102 = jnp.concatenate([context, center[:, None]],
                           axis=1).reshape(PAIRS, 2 * LC)
    rem = i102 % RW
    r102 = ((i102 // RW) * RW + ((rem & (RW // 2 - 1)) << 1)
            + (rem >> 12)).astype(jnp.int32)
    pad0 = jnp.broadcast_to(r102[:, L:LC], (PAIRS, NH - LC))
    pad1 = jnp.broadcast_to(r102[:, 2 * LC - 1:], (PAIRS, NH - LC))
    rows = jnp.concatenate(
        [r102[:, :LC], pad0,
         r102[:, LC:], pad1],
        axis=1)
    idx = jnp.pad(rows, ((0, 0), (0, 128 - 2 * NH)))
    table3 = _repack(embedding_weights.T).reshape(2 * NSLOT, D)
    packed = _krembed(idx, table3)
    return packed.reshape(BATCH, D)


# speedup vs baseline: 1.8059x; 1.0251x over previous
import functools

import jax
import jax.numpy as jnp
from jax import lax
from jax.experimental import pallas as pl
from jax.experimental.pallas import tpu as pltpu
from jax.experimental.pallas import tpu_sc as plsc

VOCAB = 1000000
D = 64
L = 50
LC = 51
NH = 56
NLANE = 16
NC = 2
NS = 16
NW = NC * NS
BATCH = 16384
PAIRS = BATCH // 2
PPT = PAIRS // NW
RW = 16384
RG = (VOCAB + RW - 1) // RW
NSLOT = RG * (RW // 2)


@functools.partial(
    pl.pallas_call,
    grid=(RG,),
    in_specs=[pl.BlockSpec((D, RW), lambda k: (0, k))],
    out_specs=pl.BlockSpec((RW // 2, 2 * D), lambda k: (k, 0)),
    out_shape=jax.ShapeDtypeStruct((NSLOT, 2 * D), jnp.float32),
)
def _repack(tT_ref, o_ref):
    xt = tT_ref[...].T
    o_ref[...] = jnp.concatenate([xt[: RW // 2], xt[RW // 2:]], axis=1)


def _shuf(x, perm):
    return x.at[perm].get(mode="promise_in_bounds")


def _splat_sum(x, lane):
    for r in (8, 4, 2, 1):
        x = x + _shuf(x, lane ^ r)
    return x


def _combine(R, rbase, out_v, prow, ocol):
    c = [R[rbase + L, pl.ds(NLANE * q, NLANE)] for q in range(4)]
    lane = lax.iota(jnp.int32, NLANE)
    zero = jnp.zeros((NLANE,), jnp.float32)
    acc = [zero] * 4
    wacc = zero
    for k in range(L):
        x = [R[rbase + k, pl.ds(NLANE * q, NLANE)] for q in range(4)]
        s = None
        for q in range(4):
            d = x[q] - c[q]
            s = d * d if s is None else s + d * d
        w = jnp.exp(_splat_sum(s, lane) * -0.5)
        wacc = wacc + w
        for q in range(4):
            acc[q] = acc[q] + w * x[q]
    inv = 1.0 / (wacc + 1e-8)
    for q in range(4):
        out_v[prow, pl.ds(ocol + NLANE * q, NLANE)] = acc[q] * inv


@functools.partial(
    pl.kernel,
    out_type=jax.ShapeDtypeStruct((PAIRS, 2 * D), jnp.float32),
    mesh=plsc.VectorSubcoreMesh(core_axis_name="c", subcore_axis_name="s"),
    compiler_params=pltpu.CompilerParams(use_tc_tiling_on_sc=False),
    scratch_types=(
        [pltpu.VMEM((PPT, 128), jnp.int32),
         pltpu.VMEM((PPT, 2 * D), jnp.float32)]
        + [pltpu.VMEM((NH, D), jnp.float32)] * 8
        + [pltpu.SemaphoreType.DMA] * 16
    ),
)
def _krembed(idx_hbm, table_hbm, out_hbm, idx_v, out_v, *bufsems):
    wid = lax.axis_index("s") * NC + lax.axis_index("c")
    base = wid * PPT
    bufs = bufsems[:8]
    semA = bufsems[8:16]
    semB = bufsems[16:]
    NBUF = 8
    NU = 2 * PPT
    pltpu.sync_copy(idx_hbm.at[pl.ds(base, PPT)], idx_v)

    def fire(u, t):
        p, off = u >> 1, (t & 1) * NH
        pltpu.async_copy(table_hbm.at[idx_v.at[p, pl.ds(off, 24)]],
                         bufs[t].at[pl.ds(0, 24)], semA[t])
        pltpu.async_copy(table_hbm.at[idx_v.at[p, pl.ds(off + 24, 32)]],
                         bufs[t].at[pl.ds(24, 32)], semB[t])

    def drain(t):
        pltpu.make_async_copy(table_hbm.at[pl.ds(0, 24)],
                              bufs[t].at[pl.ds(0, 24)], semA[t]).wait()
        pltpu.make_async_copy(table_hbm.at[pl.ds(0, 32)],
                              bufs[t].at[pl.ds(24, 32)], semB[t]).wait()

    for t in range(NBUF - 1):
        fire(t, t)

    def body(i, carry):
        ub = i * NBUF
        for t in range(NBUF):
            u = ub + t

            @pl.when(u + NBUF - 1 < NU)
            def _(u=u, t=t):
                fire(u + NBUF - 1, (t + NBUF - 1) % NBUF)

            drain(t)
            _combine(bufs[t], 0, out_v, u >> 1, (t & 1) * D)
        return carry

    lax.fori_loop(0, NU // NBUF, body, 0)
    pltpu.sync_copy(out_v, out_hbm.at[pl.ds(base, PPT)])


def kernel(context, center, embedding_weights):
    i102 = jnp.concatenate([context, center[:, None]],
                           axis=1).reshape(PAIRS, 2 * LC)
    rem = i102 % RW
    r102 = ((i102 // RW) * RW + ((rem & (RW // 2 - 1)) << 1)
            + (rem >> 13)).astype(jnp.int32)
    pad0 = jnp.broadcast_to(r102[:, L:LC], (PAIRS, NH - LC))
    pad1 = jnp.broadcast_to(r102[:, 2 * LC - 1:], (PAIRS, NH - LC))
    rows = jnp.concatenate(
        [r102[:, :LC], pad0,
         r102[:, LC:], pad1],
        axis=1)
    idx = jnp.pad(rows, ((0, 0), (0, 128 - 2 * NH)))
    table3 = _repack(embedding_weights.T).reshape(2 * NSLOT, D)
    packed = _krembed(idx, table3)
    return packed.reshape(BATCH, D)


# speedup vs baseline: 2.4869x; 1.3771x over previous
import functools

import jax
import jax.numpy as jnp
from jax import lax
from jax.experimental import pallas as pl
from jax.experimental.pallas import tpu as pltpu
from jax.experimental.pallas import tpu_sc as plsc

VOCAB = 1000000
D = 64
L = 50
LC = 51
NH = 56
NLANE = 16
NC = 2
NS = 16
NW = NC * NS
BATCH = 16384
PAIRS = BATCH // 2
PPT = PAIRS // NW
RW = 16384
RG = (VOCAB + RW - 1) // RW
NSLOT = RG * (RW // 2)


@functools.partial(
    pl.pallas_call,
    grid=(RG,),
    in_specs=[pl.BlockSpec((D, RW), lambda k: (0, k))],
    out_specs=pl.BlockSpec((RW // 2, 2 * D), lambda k: (k, 0)),
    out_shape=jax.ShapeDtypeStruct((NSLOT, 2 * D), jnp.float32),
)
def _repack(tT_ref, o_ref):
    xt = tT_ref[...].T
    o_ref[...] = jnp.concatenate([xt[: RW // 2], xt[RW // 2:]], axis=1)


def _shuf(x, perm):
    return x.at[perm].get(mode="promise_in_bounds")


def _splat_sum(x, lane):
    for r in (8, 4, 2, 1):
        x = x + _shuf(x, lane ^ r)
    return x


def _combine(R, rbase, out_v, prow, ocol):
    c = [R[rbase + L, pl.ds(NLANE * q, NLANE)] for q in range(4)]
    lane = lax.iota(jnp.int32, NLANE)
    zero = jnp.zeros((NLANE,), jnp.float32)
    acc = [zero] * 4
    wacc = zero
    for k in range(L):
        x = [R[rbase + k, pl.ds(NLANE * q, NLANE)] for q in range(4)]
        s = None
        for q in range(4):
            d = x[q] - c[q]
            s = d * d if s is None else s + d * d
        w = jnp.exp(_splat_sum(s, lane) * -0.5)
        wacc = wacc + w
        for q in range(4):
            acc[q] = acc[q] + w * x[q]
    inv = 1.0 / (wacc + 1e-8)
    for q in range(4):
        out_v[prow, pl.ds(ocol + NLANE * q, NLANE)] = acc[q] * inv


@functools.partial(
    pl.kernel,
    out_type=jax.ShapeDtypeStruct((PAIRS, 2 * D), jnp.float32),
    mesh=plsc.VectorSubcoreMesh(core_axis_name="c", subcore_axis_name="s"),
    compiler_params=pltpu.CompilerParams(use_tc_tiling_on_sc=False),
    scratch_types=(
        [pltpu.VMEM((PPT, 128), jnp.int32),
         pltpu.VMEM((PPT, 2 * D), jnp.float32)]
        + [pltpu.VMEM((NH, D), jnp.float32)] * 2
        + [pltpu.SemaphoreType.DMA] * 2
    ),
)
def _krembed(idx_hbm, table_hbm, out_hbm, idx_v, out_v, *bufsems):
    wid = lax.axis_index("s") * NC + lax.axis_index("c")
    base = wid * PPT
    bufs = bufsems[:2]
    sems = bufsems[2:]
    NU = 2 * PPT
    pltpu.sync_copy(idx_hbm.at[pl.ds(base, PPT)], idx_v)

    def fire(u, t):
        p, off = u >> 1, (t & 1) * NH
        pltpu.async_copy(table_hbm.at[idx_v.at[p, pl.ds(off, NH)]],
                         bufs[t], sems[t])

    def drain(t):
        pltpu.make_async_copy(table_hbm.at[pl.ds(0, NH)],
                              bufs[t], sems[t]).wait()

    fire(0, 0)

    def body(i, carry):
        u = i * 2
        fire(u + 1, 1)
        drain(0)
        _combine(bufs[0], 0, out_v, u >> 1, 0)

        @pl.when(u + 2 < NU)
        def _():
            fire(u + 2, 0)

        drain(1)
        _combine(bufs[1], 0, out_v, u >> 1, D)
        return carry

    lax.fori_loop(0, NU // 2, body, 0)
    pltpu.sync_copy(out_v, out_hbm.at[pl.ds(base, PPT)])


def kernel(context, center, embedding_weights):
    i102 = jnp.concatenate([context, center[:, None]],
                           axis=1).reshape(PAIRS, 2 * LC)
    rem = i102 % RW
    r102 = ((i102 // RW) * RW + ((rem & (RW // 2 - 1)) << 1)
            + (rem >> 13)).astype(jnp.int32)
    pad0 = jnp.broadcast_to(r102[:, L:LC], (PAIRS, NH - LC))
    pad1 = jnp.broadcast_to(r102[:, 2 * LC - 1:], (PAIRS, NH - LC))
    rows = jnp.concatenate(
        [r102[:, :LC], pad0,
         r102[:, LC:], pad1],
        axis=1)
    idx = jnp.pad(rows, ((0, 0), (0, 128 - 2 * NH)))
    table3 = _repack(embedding_weights.T).reshape(2 * NSLOT, D)
    packed = _krembed(idx, table3)
    return packed.reshape(BATCH, D)


# speedup vs baseline: 2.5487x; 1.0249x over previous
import functools

import jax
import jax.numpy as jnp
from jax import lax
from jax.experimental import pallas as pl
from jax.experimental.pallas import tpu as pltpu
from jax.experimental.pallas import tpu_sc as plsc

VOCAB = 1000000
D = 64
L = 50
LC = 51
NH = 56
NLANE = 16
NC = 2
NS = 16
NW = NC * NS
BATCH = 16384
PAIRS = BATCH // 2
PPT = PAIRS // NW
RW = 32768
RG = (VOCAB + RW - 1) // RW
NSLOT = RG * (RW // 2)


@functools.partial(
    pl.pallas_call,
    grid=(RG,),
    in_specs=[pl.BlockSpec((D, RW), lambda k: (0, k))],
    out_specs=pl.BlockSpec((RW // 2, 2 * D), lambda k: (k, 0)),
    out_shape=jax.ShapeDtypeStruct((NSLOT, 2 * D), jnp.float32),
)
def _repack(tT_ref, o_ref):
    xt = tT_ref[...].T
    o_ref[...] = jnp.concatenate([xt[: RW // 2], xt[RW // 2:]], axis=1)


def _shuf(x, perm):
    return x.at[perm].get(mode="promise_in_bounds")


def _splat_sum(x, lane):
    for r in (8, 4, 2, 1):
        x = x + _shuf(x, lane ^ r)
    return x


def _combine(R, rbase, out_v, prow, ocol):
    c = [R[rbase + L, pl.ds(NLANE * q, NLANE)] for q in range(4)]
    lane = lax.iota(jnp.int32, NLANE)
    zero = jnp.zeros((NLANE,), jnp.float32)
    acc = [zero] * 4
    wacc = zero
    for k in range(L):
        x = [R[rbase + k, pl.ds(NLANE * q, NLANE)] for q in range(4)]
        s = None
        for q in range(4):
            d = x[q] - c[q]
            s = d * d if s is None else s + d * d
        w = jnp.exp(_splat_sum(s, lane) * -0.5)
        wacc = wacc + w
        for q in range(4):
            acc[q] = acc[q] + w * x[q]
    inv = 1.0 / (wacc + 1e-8)
    for q in range(4):
        out_v[prow, pl.ds(ocol + NLANE * q, NLANE)] = acc[q] * inv


@functools.partial(
    pl.kernel,
    out_type=jax.ShapeDtypeStruct((PAIRS, 2 * D), jnp.float32),
    mesh=plsc.VectorSubcoreMesh(core_axis_name="c", subcore_axis_name="s"),
    compiler_params=pltpu.CompilerParams(use_tc_tiling_on_sc=False),
    scratch_types=(
        [pltpu.VMEM((PPT, 128), jnp.int32),
         pltpu.VMEM((PPT, 2 * D), jnp.float32)]
        + [pltpu.VMEM((NH, D), jnp.float32)] * 2
        + [pltpu.SemaphoreType.DMA] * 2
    ),
)
def _krembed(idx_hbm, table_hbm, out_hbm, idx_v, out_v, *bufsems):
    wid = lax.axis_index("s") * NC + lax.axis_index("c")
    base = wid * PPT
    bufs = bufsems[:2]
    sems = bufsems[2:]
    NU = 2 * PPT
    pltpu.sync_copy(idx_hbm.at[pl.ds(base, PPT)], idx_v)

    def fire(u, t):
        p, off = u >> 1, (t & 1) * NH
        pltpu.async_copy(table_hbm.at[idx_v.at[p, pl.ds(off, NH)]],
                         bufs[t], sems[t])

    def drain(t):
        pltpu.make_async_copy(table_hbm.at[pl.ds(0, NH)],
                              bufs[t], sems[t]).wait()

    fire(0, 0)

    def body(i, carry):
        u = i * 2
        fire(u + 1, 1)
        drain(0)
        _combine(bufs[0], 0, out_v, u >> 1, 0)

        @pl.when(u + 2 < NU)
        def _():
            fire(u + 2, 0)

        drain(1)
        _combine(bufs[1], 0, out_v, u >> 1, D)
        return carry

    lax.fori_loop(0, NU // 2, body, 0)
    pltpu.sync_copy(out_v, out_hbm.at[pl.ds(base, PPT)])


def kernel(context, center, embedding_weights):
    i102 = jnp.concatenate([context, center[:, None]],
                           axis=1).reshape(PAIRS, 2 * LC)
    rem = i102 % RW
    r102 = ((i102 // RW) * RW + ((rem & (RW // 2 - 1)) << 1)
            + (rem >> 14)).astype(jnp.int32)
    pad0 = jnp.broadcast_to(r102[:, L:LC], (PAIRS, NH - LC))
    pad1 = jnp.broadcast_to(r102[:, 2 * LC - 1:], (PAIRS, NH - LC))
    rows = jnp.concatenate(
        [r102[:, :LC], pad0,
         r102[:, LC:], pad1],
        axis=1)
    idx = jnp.pad(rows, ((0, 0), (0, 128 - 2 * NH)))
    table3 = _repack(embedding_weights.T).reshape(2 * NSLOT, D)
    packed = _krembed(idx, table3)
    return packed.reshape(BATCH, D)
